# baseline (speedup 1.0000x reference)
import jax
import jax.numpy as jnp
from jax import lax
from jax.experimental import pallas as pl
from jax.experimental.pallas import tpu as pltpu
from jax.experimental.pallas import tpu_sc as plsc

N_NODES = 50000
N_EDGES = 800000
NBLK = N_EDGES // 128
BLK_LO = NBLK // 32
BLK_HI = BLK_LO + 1
CHUNK = 3200
NFULL = 7
TAIL_HI = BLK_HI * 128 - NFULL * CHUNK
TAIL_LO = BLK_LO * 128 - NFULL * CHUNK
P2_LO = NBLK // 16
P2_HI = P2_LO + 1
NFULL2 = 15
TAIL2_HI = P2_HI * 128 - NFULL2 * CHUNK
TAIL2_LO = P2_LO * 128 - NFULL2 * CHUNK
N_ACC = 51200
ZSLICE = N_ACC // 16
L = 16


def _rowsum_T(m):
    ones = jnp.ones((1, m.shape[1]), jnp.float32)
    dn = (((1,), (1,)), ((), ()))
    m_hi = m.astype(jnp.bfloat16).astype(jnp.float32)
    m_lo = m - m_hi
    hi = jax.lax.dot_general(ones, m_hi, dn, preferred_element_type=jnp.float32)
    lo = jax.lax.dot_general(ones, m_lo, dn, preferred_element_type=jnp.float32)
    return (hi + lo)[0]


def _s0_body(x_ref, o_ref, os_ref):
    x = x_ref[...]
    rs = _rowsum_T(x)
    sq = _rowsum_T(x * x)
    s0 = rs / jnp.maximum(jnp.sqrt(sq), 1e-12)
    o_ref[...] = s0
    os_ref[...] = jnp.reshape(jnp.sum(s0), (1, 1))


def _compute_s0(emb):
    return pl.pallas_call(
        _s0_body,
        out_shape=(
            jax.ShapeDtypeStruct((N_NODES,), jnp.float32),
            jax.ShapeDtypeStruct((1, 1), jnp.float32),
        ),
    )(emb)


def _edge_body(adj_hbm, w_hbm, s0_hbm, t2p_hbm, s1p_hbm,
               s0_v, ed_v, src_v, dst_v, w_v, msg_v, acc_v, sum_v, s1_sh):
    core = lax.axis_index("c")
    sid = lax.axis_index("s")
    wid = sid * 2 + core
    zeros16 = jnp.zeros((L,), jnp.float32)

    def zbody(i, _):
        msg_v[pl.ds(i * L, L)] = zeros16
        return 0
    lax.fori_loop(0, CHUNK // L, zbody, 0)

    zs = pl.ds(sid * ZSLICE, ZSLICE)
    pltpu.sync_copy(msg_v, s1_sh.at[zs])

    pltpu.sync_copy(s0_hbm, s0_v.at[pl.ds(0, N_NODES)])
    plsc.subcore_barrier()

    base1 = (wid * BLK_LO + jnp.minimum(wid, 10)) * 128

    def chunk1(off, nedge):
        pltpu.sync_copy(adj_hbm.at[:, pl.ds(off, nedge)], ed_v.at[:, pl.ds(0, nedge)])
        pltpu.sync_copy(w_hbm.at[pl.ds(off, nedge)], w_v.at[pl.ds(0, nedge)])

        def gbody(i, _):
            o = i * L
            sv = ed_v[0, pl.ds(o, L)]
            dv = ed_v[1, pl.ds(o, L)]
            src_v[pl.ds(o, L)] = sv
            dst_v[pl.ds(o, L)] = dv
            vals = plsc.load_gather(s0_v, [sv])
            msg_v[pl.ds(o, L)] = w_v[pl.ds(o, L)] * vals
            return 0
        lax.fori_loop(0, nedge // L, gbody, 0)

        pltpu.sync_copy(msg_v.at[pl.ds(0, nedge)],
                        s1_sh.at[dst_v.at[pl.ds(0, nedge)]], add=True)

    def c1body(c, _):
        chunk1(base1 + c * CHUNK, CHUNK)
        return 0
    lax.fori_loop(0, NFULL, c1body, 0)

    tail1 = base1 + NFULL * CHUNK

    @pl.when(wid < 10)
    def _t1hi():
        chunk1(tail1, TAIL_HI)

    @pl.when(wid >= 10)
    def _t1lo():
        chunk1(tail1, TAIL_LO)

    plsc.subcore_barrier()

    pltpu.sync_copy(s1_sh, s0_v)

    def sbody(i, a):
        return a + s0_v[pl.ds(sid * ZSLICE + i * L, L)]
    sum_v[...] = lax.fori_loop(0, ZSLICE // L, sbody, zeros16)

    base2 = (sid * P2_LO + jnp.minimum(sid, 10)) * 128

    def chunk2(off, nedge, acc):
        pltpu.sync_copy(adj_hbm.at[:, pl.ds(off, nedge)], ed_v.at[:, pl.ds(0, nedge)])
        pltpu.sync_copy(w_hbm.at[pl.ds(off, nedge)], w_v.at[pl.ds(0, nedge)])

        def gbody(i, a):
            o = i * L
            sv = ed_v[0, pl.ds(o, L)]
            vals = plsc.load_gather(s0_v, [sv])
            return a + w_v[pl.ds(o, L)] * vals
        return lax.fori_loop(0, nedge // L, gbody, acc)

    def c2body(c, acc):
        return chunk2(base2 + c * CHUNK, CHUNK, acc)
    acc_v[...] = lax.fori_loop(0, NFULL2, c2body, zeros16)

    tail2 = base2 + NFULL2 * CHUNK

    @pl.when(sid < 10)
    def _t2hi():
        acc_v[...] = chunk2(tail2, TAIL2_HI, acc_v[...])

    @pl.when(sid >= 10)
    def _t2lo():
        acc_v[...] = chunk2(tail2, TAIL2_LO, acc_v[...])

    pltpu.sync_copy(acc_v, t2p_hbm.at[pl.ds(wid * L, L)])
    pltpu.sync_copy(sum_v, s1p_hbm.at[pl.ds(wid * L, L)])


def _edge_pass(adj, w, s0):
    mesh = plsc.VectorSubcoreMesh(core_axis_name="c", subcore_axis_name="s")
    f = pl.kernel(
        _edge_body,
        out_type=(
            jax.ShapeDtypeStruct((32 * L,), jnp.float32),
            jax.ShapeDtypeStruct((32 * L,), jnp.float32),
        ),
        mesh=mesh,
        compiler_params=pltpu.CompilerParams(needs_layout_passes=False),
        scratch_types=[
            pltpu.VMEM((N_ACC,), jnp.float32),
            pltpu.VMEM((2, CHUNK), jnp.int32),
            pltpu.VMEM((CHUNK,), jnp.int32),
            pltpu.VMEM((CHUNK,), jnp.int32),
            pltpu.VMEM((CHUNK,), jnp.float32),
            pltpu.VMEM((CHUNK,), jnp.float32),
            pltpu.VMEM((L,), jnp.float32),
            pltpu.VMEM((L,), jnp.float32),
            pltpu.VMEM_SHARED((N_ACC,), jnp.float32),
        ],
    )
    return f(adj, w, s0)


def kernel(adj, weight_vector, id_embedding):
    w = weight_vector[:, 0]
    s0, s0sum = _compute_s0(id_embedding)
    t2p, s1p = _edge_pass(adj, w, s0)
    return s0sum[0, 0] + jnp.sum(s1p) + jnp.sum(t2p)


# speedup vs baseline: 1.5216x; 1.5216x over previous
import jax
import jax.numpy as jnp
from jax import lax
from jax.experimental import pallas as pl
from jax.experimental.pallas import tpu as pltpu
from jax.experimental.pallas import tpu_sc as plsc

N_NODES = 50000
N_EDGES = 800000
NBLK = N_EDGES // 128
BLK_LO = NBLK // 32
BLK_HI = BLK_LO + 1
CHUNK = 3200
NFULL = 7
TAIL_HI = BLK_HI * 128 - NFULL * CHUNK
TAIL_LO = BLK_LO * 128 - NFULL * CHUNK
N_ACC = 50048
ZSLICE = N_ACC // 16
ROW_BLOCK = 1024
L = 16


def _rowsum_T(m):
    ones = jnp.ones((1, m.shape[1]), jnp.float32)
    dn = (((1,), (1,)), ((), ()))
    m_hi = m.astype(jnp.bfloat16).astype(jnp.float32)
    m_lo = m - m_hi
    hi = jax.lax.dot_general(ones, m_hi, dn, preferred_element_type=jnp.float32)
    lo = jax.lax.dot_general(ones, m_lo, dn, preferred_element_type=jnp.float32)
    return (hi + lo)[0]


def _s0_body(x_ref, o_ref):
    x = x_ref[...]
    rs = _rowsum_T(x)
    sq = _rowsum_T(x * x)
    o_ref[...] = rs / jnp.maximum(jnp.sqrt(sq), 1e-12)


def _compute_s0(emb):
    return pl.pallas_call(
        _s0_body,
        out_shape=jax.ShapeDtypeStruct((N_NODES,), jnp.float32),
    )(emb)


def _edge_body(adj_hbm, w_hbm, s0_hbm, s1p0_hbm, s1p1_hbm, ap0_hbm, ap1_hbm,
               s0_v,
               ed0, ed1, ed2, src0, src1, src2, dst0, dst1, dst2,
               w0, w1, w2, m0, m1, m2,
               s1_sh, a_sh,
               lsem0, lsem1, lsem2, ssem0, ssem1, ssem2):
    edb = [ed0, ed1, ed2]
    srcb = [src0, src1, src2]
    dstb = [dst0, dst1, dst2]
    wb = [w0, w1, w2]
    mb = [m0, m1, m2]
    lsem = [lsem0, lsem1, lsem2]
    ssem = [ssem0, ssem1, ssem2]

    core = lax.axis_index("c")
    sid = lax.axis_index("s")
    wid = sid * 2 + core

    def zbody(i, _):
        m0[pl.ds(i * L, L)] = jnp.zeros((L,), jnp.float32)
        return 0
    lax.fori_loop(0, CHUNK // L, zbody, 0)

    zs = pl.ds(sid * ZSLICE, ZSLICE)
    pltpu.sync_copy(m0.at[pl.ds(0, ZSLICE)], s1_sh.at[zs])
    pltpu.sync_copy(m0.at[pl.ds(0, ZSLICE)], a_sh.at[zs])

    pltpu.sync_copy(s0_hbm, s0_v)
    plsc.subcore_barrier()

    base = (wid * BLK_LO + jnp.minimum(wid, 10)) * 128

    def issue_load(c):
        s = c % 3
        off = base + c * CHUNK
        dl = pltpu.async_copy(adj_hbm.at[:, pl.ds(off, CHUNK)], edb[s], lsem[s])
        dw = pltpu.async_copy(w_hbm.at[pl.ds(off, CHUNK)], wb[s], lsem[s])
        return (dl, dw)

    def gather(s, nedge):
        def gbody(i, _):
            for u in range(4):
                o = i * 64 + u * L
                sv = edb[s][0, pl.ds(o, L)]
                dv = edb[s][1, pl.ds(o, L)]
                srcb[s][pl.ds(o, L)] = sv
                dstb[s][pl.ds(o, L)] = dv
                vals = plsc.load_gather(s0_v, [sv])
                mb[s][pl.ds(o, L)] = wb[s][pl.ds(o, L)] * vals
            return 0
        lax.fori_loop(0, nedge // 64, gbody, 0)

    def issue_scat(s, nedge):
        d1 = pltpu.async_copy(mb[s].at[pl.ds(0, nedge)],
                              s1_sh.at[dstb[s].at[pl.ds(0, nedge)]],
                              ssem[s], add=True)
        d2 = pltpu.async_copy(wb[s].at[pl.ds(0, nedge)],
                              a_sh.at[srcb[s].at[pl.ds(0, nedge)]],
                              ssem[s], add=True)
        return (d1, d2)

    loads = {0: issue_load(0)}
    scats = {}
    for c in range(NFULL):
        if c + 1 < NFULL:
            if c - 2 >= 0:
                for d in scats[c - 2]:
                    d.wait()
            loads[c + 1] = issue_load(c + 1)
        for d in loads[c]:
            d.wait()
        gather(c % 3, CHUNK)
        scats[c] = issue_scat(c % 3, CHUNK)

    for d in scats[NFULL - 3]:
        d.wait()
    ts = NFULL % 3
    tail_off = base + NFULL * CHUNK

    def do_tail(nedge):
        pltpu.sync_copy(adj_hbm.at[:, pl.ds(tail_off, nedge)],
                        edb[ts].at[:, pl.ds(0, nedge)])
        pltpu.sync_copy(w_hbm.at[pl.ds(tail_off, nedge)],
                        wb[ts].at[pl.ds(0, nedge)])
        gather(ts, nedge)
        pltpu.sync_copy(mb[ts].at[pl.ds(0, nedge)],
                        s1_sh.at[dstb[ts].at[pl.ds(0, nedge)]], add=True)
        pltpu.sync_copy(wb[ts].at[pl.ds(0, nedge)],
                        a_sh.at[srcb[ts].at[pl.ds(0, nedge)]], add=True)

    @pl.when(wid < 10)
    def _tail_hi():
        do_tail(TAIL_HI)

    @pl.when(wid >= 10)
    def _tail_lo():
        do_tail(TAIL_LO)

    for d in scats[NFULL - 2]:
        d.wait()
    for d in scats[NFULL - 1]:
        d.wait()

    plsc.subcore_barrier()

    pltpu.sync_copy(s1_sh.at[zs], m0.at[pl.ds(0, ZSLICE)])
    pltpu.sync_copy(a_sh.at[zs], w0.at[pl.ds(0, ZSLICE)])

    @pl.when(core == 0)
    def _pub0():
        pltpu.sync_copy(m0.at[pl.ds(0, ZSLICE)], s1p0_hbm.at[zs])
        pltpu.sync_copy(w0.at[pl.ds(0, ZSLICE)], ap0_hbm.at[zs])

    @pl.when(core == 1)
    def _pub1():
        pltpu.sync_copy(m0.at[pl.ds(0, ZSLICE)], s1p1_hbm.at[zs])
        pltpu.sync_copy(w0.at[pl.ds(0, ZSLICE)], ap1_hbm.at[zs])


def _edge_pass(adj, w, s0):
    mesh = plsc.VectorSubcoreMesh(core_axis_name="c", subcore_axis_name="s")
    f = pl.kernel(
        _edge_body,
        out_type=(
            jax.ShapeDtypeStruct((N_ACC,), jnp.float32),
            jax.ShapeDtypeStruct((N_ACC,), jnp.float32),
            jax.ShapeDtypeStruct((N_ACC,), jnp.float32),
            jax.ShapeDtypeStruct((N_ACC,), jnp.float32),
        ),
        mesh=mesh,
        compiler_params=pltpu.CompilerParams(needs_layout_passes=False),
        scratch_types=(
            [pltpu.VMEM((N_NODES,), jnp.float32)]
            + [pltpu.VMEM((2, CHUNK), jnp.int32) for _ in range(3)]
            + [pltpu.VMEM((CHUNK,), jnp.int32) for _ in range(6)]
            + [pltpu.VMEM((CHUNK,), jnp.float32) for _ in range(6)]
            + [pltpu.VMEM_SHARED((N_ACC,), jnp.float32) for _ in range(2)]
            + [pltpu.SemaphoreType.DMA for _ in range(6)]
        ),
    )
    return f(adj, w, s0)


def _final_body(s0_ref, s10_ref, s11_ref, a0_ref, a1_ref, o_ref):
    s0 = s0_ref[...]
    s1 = s10_ref[...] + s11_ref[...]
    a = a0_ref[...] + a1_ref[...]
    tot = jnp.sum(s0) + jnp.sum(s1) + jnp.sum(s1 * a)
    o_ref[...] = jnp.reshape(tot, (1, 1))


def _final_reduce(s0, s1p0, s1p1, ap0, ap1):
    return pl.pallas_call(
        _final_body,
        out_shape=jax.ShapeDtypeStruct((1, 1), jnp.float32),
    )(s0, s1p0, s1p1, ap0, ap1)


def kernel(adj, weight_vector, id_embedding):
    w = weight_vector[:, 0]
    s0 = _compute_s0(id_embedding)
    s1p0, s1p1, ap0, ap1 = _edge_pass(adj, w, s0)
    out = _final_reduce(s0, s1p0, s1p1, ap0, ap1)
    return out[0, 0]
